# Initial kernel scaffold; baseline (speedup 1.0000x reference)
import jax
import jax.numpy as jnp
from jax.experimental import pallas as pl


def kernel(input, adj_edge_index, adj_values, h0, W, lth):
    raise NotImplementedError("write your pallas kernel here")


# speedup vs baseline: 5.2805x; 5.2805x over previous
import functools
import math

import jax
import jax.numpy as jnp
from jax import lax
from jax.experimental import pallas as pl
from jax.experimental.pallas import tpu as pltpu
from jax.experimental.pallas import tpu_sc as plsc

ALPHA = 0.1
THETA = 0.5
BETA = math.log(THETA / 2 + 1.0)

NC = 2
NS = 16
NW = NC * NS
L = 16
C = 128


def _sc_spmm_kernel(N, D, nchunk):
    rpw = N // NS
    zr = 125
    assert rpw % zr == 0

    mesh = plsc.VectorSubcoreMesh(core_axis_name="c", subcore_axis_name="s")

    @functools.partial(
        pl.kernel,
        out_type=jax.ShapeDtypeStruct((NC, N, D), jnp.float32),
        mesh=mesh,
        scratch_types=[
            pltpu.VMEM((nchunk, C), jnp.int32),
            pltpu.VMEM((nchunk, C), jnp.int32),
            pltpu.VMEM((nchunk, C), jnp.float32),
            pltpu.VMEM((C, D), jnp.float32),
            pltpu.VMEM_SHARED((N, D), jnp.float32),
            pltpu.SemaphoreType.DMA,
        ],
    )
    def spmm(x_hbm, src_hbm, dst_hbm, val_hbm, out_hbm,
             srcv, dstv, valv, rows, hacc, sem):
        cid = lax.axis_index("c")
        sid = lax.axis_index("s")
        wid = cid * NS + sid
        zeros = jnp.zeros((L,), jnp.float32)

        def zrow(i, carry):
            for j in range(D // L):
                rows[i, pl.ds(L * j, L)] = zeros
            return carry
        lax.fori_loop(0, zr, zrow, 0)
        for k in range(rpw // zr):
            pltpu.sync_copy(rows.at[pl.ds(0, zr)],
                            hacc.at[pl.ds(sid * rpw + k * zr, zr)])
        plsc.subcore_barrier()

        pltpu.sync_copy(src_hbm.at[wid], srcv)
        pltpu.sync_copy(dst_hbm.at[wid], dstv)
        pltpu.sync_copy(val_hbm.at[wid], valv)

        def chunk_body(t, carry):
            pltpu.async_copy(x_hbm.at[srcv.at[t]], rows, sem).wait()

            def scale_grp(g, c2):
                vv = valv[t, pl.ds(g * L, L)]
                for ri in range(L):
                    v = jnp.full((L,), vv[ri])
                    r = g * L + ri
                    for j in range(D // L):
                        rows[r, pl.ds(L * j, L)] = rows[r, pl.ds(L * j, L)] * v
                return c2
            lax.fori_loop(0, C // L, scale_grp, 0)
            pltpu.sync_copy(rows, hacc.at[dstv.at[t]], add=True)
            return carry
        lax.fori_loop(0, nchunk, chunk_body, 0)
        plsc.subcore_barrier()

        @pl.when(sid == 0)
        def _():
            pltpu.sync_copy(hacc, out_hbm.at[cid])

    return spmm


def _tc_epilogue(hp, h0, W):
    N, D = h0.shape
    R = 2000
    assert N % R == 0

    def body(hp_ref, h0_ref, w_ref, out_ref):
        h = (hp_ref[0] + hp_ref[1]) * (1.0 - ALPHA)
        support = h + ALPHA * h0_ref[...]
        out_ref[...] = (
            BETA * jnp.dot(support, w_ref[...],
                           preferred_element_type=jnp.float32)
            + (1.0 - BETA) * support)

    return pl.pallas_call(
        body,
        grid=(N // R,),
        in_specs=[
            pl.BlockSpec((NC, R, D), lambda i: (0, i, 0)),
            pl.BlockSpec((R, D), lambda i: (i, 0)),
            pl.BlockSpec((D, D), lambda i: (0, 0)),
        ],
        out_specs=pl.BlockSpec((R, D), lambda i: (i, 0)),
        out_shape=jax.ShapeDtypeStruct((N, D), jnp.float32),
    )(hp, h0, W)


def kernel(input, adj_edge_index, adj_values, h0, W, lth):
    N, D = input.shape
    E = adj_values.shape[0]
    nchunk = -(-E // (NW * C))
    e_pad = NW * nchunk * C - E
    src = jnp.concatenate([adj_edge_index[0], jnp.zeros((e_pad,), jnp.int32)])
    dst = jnp.concatenate([adj_edge_index[1], jnp.zeros((e_pad,), jnp.int32)])
    vals = jnp.concatenate([adj_values, jnp.zeros((e_pad,), jnp.float32)])
    src = src.reshape(NW, nchunk, C)
    dst = dst.reshape(NW, nchunk, C)
    vals = vals.reshape(NW, nchunk, C)
    hp = _sc_spmm_kernel(N, D, nchunk)(input, src, dst, vals)
    return _tc_epilogue(hp, h0, W)
